# Initial kernel scaffold; baseline (speedup 1.0000x reference)
import jax
import jax.numpy as jnp
from jax.experimental import pallas as pl


def kernel(edge_index, emb, W1, b1, W2, b2, W3, b3):
    raise NotImplementedError("write your pallas kernel here")


# speedup vs baseline: 10.8273x; 10.8273x over previous
import functools

import jax
import jax.numpy as jnp
from jax import lax
from jax.experimental import pallas as pl
from jax.experimental.pallas import tpu as pltpu
from jax.experimental.pallas import tpu_sc as plsc

N = 10000
NP = 10240
E = 320000
H = 128
NC = 2
NS = 16
NW = NC * NS
EPT = E // NW
K = 80
NIT = EPT // K
RPT = NP // NS


def _sc_mesh():
    return plsc.VectorSubcoreMesh(core_axis_name="c", subcore_axis_name="s")


@functools.partial(
    pl.kernel,
    out_type=jax.ShapeDtypeStruct((NC * NP, 16), jnp.float32),
    mesh=_sc_mesh(),
    scratch_types=[
        pltpu.VMEM((K,), jnp.int32),
        pltpu.VMEM((K, 16), jnp.float32),
        pltpu.VMEM_SHARED((NP, 16), jnp.float32),
    ],
)
def _deg_kernel(dst_hbm, zeros_hbm, ones_hbm, out_hbm, idx_v, ones_v, acc):
    c = lax.axis_index("c")
    s = lax.axis_index("s")
    wid = c * NS + s
    pltpu.sync_copy(ones_hbm, ones_v)
    pltpu.sync_copy(zeros_hbm.at[pl.ds(s * RPT, RPT)], acc.at[pl.ds(s * RPT, RPT)])
    plsc.subcore_barrier()
    base0 = wid * EPT

    def body(j, carry):
        pltpu.sync_copy(dst_hbm.at[pl.ds(base0 + j * K, K)], idx_v)
        pltpu.sync_copy(ones_v, acc.at[idx_v], add=True)
        return carry

    lax.fori_loop(0, NIT, body, 0)
    plsc.subcore_barrier()
    pltpu.sync_copy(acc.at[pl.ds(s * RPT, RPT)],
                    out_hbm.at[pl.ds(c * NP + s * RPT, RPT)])


@functools.partial(
    pl.kernel,
    out_type=jax.ShapeDtypeStruct((NC * NP, H), jnp.float32),
    mesh=_sc_mesh(),
    scratch_types=[
        pltpu.VMEM((K,), jnp.int32),
        pltpu.VMEM((K,), jnp.int32),
        pltpu.VMEM((K, H), jnp.float32),
        pltpu.VMEM_SHARED((NP, H), jnp.float32),
        pltpu.SemaphoreType.DMA,
    ],
)
def _edge_kernel(hs_hbm, src_hbm, dst_hbm, zeros_hbm, out_hbm,
                 src_v, dst_v, rows_v, acc, sem):
    c = lax.axis_index("c")
    s = lax.axis_index("s")
    wid = c * NS + s
    pltpu.sync_copy(zeros_hbm.at[pl.ds(s * RPT, RPT)], acc.at[pl.ds(s * RPT, RPT)])
    plsc.subcore_barrier()
    base0 = wid * EPT

    def body(j, carry):
        base = base0 + j * K
        pltpu.sync_copy(src_hbm.at[pl.ds(base, K)], src_v)
        pltpu.sync_copy(dst_hbm.at[pl.ds(base, K)], dst_v)
        pltpu.async_copy(hs_hbm.at[src_v], rows_v, sem).wait()
        pltpu.sync_copy(rows_v, acc.at[dst_v], add=True)
        return carry

    lax.fori_loop(0, NIT, body, 0)
    plsc.subcore_barrier()
    pltpu.sync_copy(acc.at[pl.ds(s * RPT, RPT)],
                    out_hbm.at[pl.ds(c * NP + s * RPT, RPT)])


B = 1000


def _mm1_body(degp_ref, x_ref, w_ref, hs_ref, dinv_ref):
    deg = degp_ref[0, :, 0:1] + degp_ref[1, :, 0:1] + 1.0
    dinv = lax.rsqrt(deg)
    h = jnp.dot(x_ref[...], w_ref[...], preferred_element_type=jnp.float32)
    hs_ref[...] = dinv * h
    dinv_ref[...] = dinv


_mm1 = pl.pallas_call(
    _mm1_body,
    grid=(N // B,),
    in_specs=[
        pl.BlockSpec((2, B, 16), lambda i: (0, i, 0)),
        pl.BlockSpec((B, H), lambda i: (i, 0)),
        pl.BlockSpec((H, H), lambda i: (0, 0)),
    ],
    out_specs=[
        pl.BlockSpec((B, H), lambda i: (i, 0)),
        pl.BlockSpec((B, 1), lambda i: (i, 0)),
    ],
    out_shape=[
        jax.ShapeDtypeStruct((N, H), jnp.float32),
        jax.ShapeDtypeStruct((N, 1), jnp.float32),
    ],
)


def _mid_body(sp_ref, hs_ref, dinv_ref, b_ref, w_ref, out_ref):
    dinv = dinv_ref[...]
    t = sp_ref[0] + sp_ref[1] + hs_ref[...]
    x = jnp.maximum(dinv * t + b_ref[...], 0.0)
    out_ref[...] = dinv * jnp.dot(x, w_ref[...], preferred_element_type=jnp.float32)


_mm_mid = pl.pallas_call(
    _mid_body,
    grid=(N // B,),
    in_specs=[
        pl.BlockSpec((2, B, H), lambda i: (0, i, 0)),
        pl.BlockSpec((B, H), lambda i: (i, 0)),
        pl.BlockSpec((B, 1), lambda i: (i, 0)),
        pl.BlockSpec((1, H), lambda i: (0, 0)),
        pl.BlockSpec((H, H), lambda i: (0, 0)),
    ],
    out_specs=pl.BlockSpec((B, H), lambda i: (i, 0)),
    out_shape=jax.ShapeDtypeStruct((N, H), jnp.float32),
)


def _fin_body(sp_ref, hs_ref, dinv_ref, b_ref, out_ref):
    t = sp_ref[0] + sp_ref[1] + hs_ref[...]
    out_ref[...] = dinv_ref[...] * t + b_ref[...]


_mm_fin = pl.pallas_call(
    _fin_body,
    grid=(N // B,),
    in_specs=[
        pl.BlockSpec((2, B, H), lambda i: (0, i, 0)),
        pl.BlockSpec((B, H), lambda i: (i, 0)),
        pl.BlockSpec((B, 1), lambda i: (i, 0)),
        pl.BlockSpec((1, H), lambda i: (0, 0)),
    ],
    out_specs=pl.BlockSpec((B, H), lambda i: (i, 0)),
    out_shape=jax.ShapeDtypeStruct((N, H), jnp.float32),
)


def kernel(edge_index, emb, W1, b1, W2, b2, W3, b3):
    src = edge_index[0]
    dst = edge_index[1]
    zeros16 = jnp.zeros((NP, 16), jnp.float32)
    zerosH = jnp.zeros((NP, H), jnp.float32)
    ones = jnp.ones((K, 16), jnp.float32)

    degp = _deg_kernel(dst, zeros16, ones).reshape(2, NP, 16)[:, :N, :]
    hs1, dinv = _mm1(degp, emb, W1)
    s1 = _edge_kernel(hs1, src, dst, zerosH).reshape(2, NP, H)[:, :N, :]
    hs2 = _mm_mid(s1, hs1, dinv, b1.reshape(1, H), W2)
    s2 = _edge_kernel(hs2, src, dst, zerosH).reshape(2, NP, H)[:, :N, :]
    hs3 = _mm_mid(s2, hs2, dinv, b2.reshape(1, H), W3)
    s3 = _edge_kernel(hs3, src, dst, zerosH).reshape(2, NP, H)[:, :N, :]
    out = _mm_fin(s3, hs3, dinv, b3.reshape(1, H))
    return out
